# Initial kernel scaffold; baseline (speedup 1.0000x reference)
import functools

import jax
import jax.numpy as jnp
from jax import lax
from jax.experimental import pallas as pl
from jax.experimental.pallas import tpu as pltpu
from jax.experimental.pallas import tpu_sc as plsc

NC = 2
NS = 16
NW = NC * NS
L = 16
CH = 128


def _pad_up(x, m):
    return (x + m - 1) // m * m


def _make_spmm(n_acc, h, chunks):
    rows_ps = n_acc // NS
    mesh = plsc.VectorSubcoreMesh(core_axis_name="c", subcore_axis_name="s")

    @functools.partial(
        pl.kernel,
        out_type=jax.ShapeDtypeStruct((NC, n_acc, h), jnp.float32),
        mesh=mesh,
        scratch_types=[
            pltpu.VMEM((chunks, CH), jnp.int32),
            pltpu.VMEM((chunks, CH), jnp.int32),
            pltpu.VMEM((CH, h), jnp.float32),
            pltpu.VMEM((rows_ps, h), jnp.float32),
            pltpu.VMEM_SHARED((n_acc, h), jnp.float32),
            pltpu.SemaphoreType.DMA,
        ],
    )
    def spmm(table_hbm, src_hbm, dst_hbm, out_hbm,
             src_v, dst_v, rows_v, stage_v, acc_sh, sem):
        cid = lax.axis_index("c")
        sid = lax.axis_index("s")
        wid = sid * NC + cid

        def _z(i, c):
            stage_v[i, :] = jnp.zeros((L,), jnp.float32)
            return c
        lax.fori_loop(0, rows_ps, _z, 0)
        pltpu.sync_copy(stage_v, acc_sh.at[pl.ds(sid * rows_ps, rows_ps)])

        pltpu.sync_copy(src_hbm.at[wid], src_v)
        pltpu.sync_copy(dst_hbm.at[wid], dst_v)
        plsc.subcore_barrier()

        def _chunk(j, c):
            pltpu.async_copy(table_hbm.at[src_v.at[j]], rows_v, sem).wait()
            pltpu.sync_copy(rows_v, acc_sh.at[dst_v.at[j]], add=True)
            return c
        lax.fori_loop(0, chunks, _chunk, 0)
        plsc.subcore_barrier()

        pltpu.sync_copy(acc_sh.at[pl.ds(sid * rows_ps, rows_ps)], stage_v)
        pltpu.sync_copy(stage_v, out_hbm.at[cid, pl.ds(sid * rows_ps, rows_ps)])

    return spmm


def _mm(x, w):
    def body(x_ref, w_ref, o_ref):
        o_ref[...] = jnp.dot(x_ref[...], w_ref[...],
                             preferred_element_type=jnp.float32)
    return pl.pallas_call(
        body,
        out_shape=jax.ShapeDtypeStruct((x.shape[0], w.shape[1]), jnp.float32),
    )(x, w)


def _combine_relu(p):
    def body(p_ref, o_ref):
        o_ref[...] = jnp.maximum(p_ref[0] + p_ref[1], 0.0)
    return pl.pallas_call(
        body,
        out_shape=jax.ShapeDtypeStruct(p.shape[1:], jnp.float32),
    )(p)


def _combine_mm(q, w):
    def body(q_ref, w_ref, o_ref):
        o_ref[...] = jnp.dot(q_ref[0] + q_ref[1], w_ref[...],
                             preferred_element_type=jnp.float32)
    return pl.pallas_call(
        body,
        out_shape=jax.ShapeDtypeStruct((q.shape[1], w.shape[1]), jnp.float32),
    )(q, w)


def kernel(features, edge_index, W1, W3):
    n, d = features.shape
    h = W1.shape[1]
    e = edge_index.shape[1]

    n_acc = _pad_up(n + 1, NS * 8)
    chunks = _pad_up(e, NW * CH) // (NW * CH)
    e_pad = NW * chunks * CH

    fp = jnp.pad(features, ((0, n_acc - n), (0, 0)))
    pad = jnp.full((e_pad - e,), n, jnp.int32)
    src = jnp.concatenate([edge_index[0], pad]).reshape(NW, chunks, CH)
    dst = jnp.concatenate([edge_index[1], pad]).reshape(NW, chunks, CH)

    spmm = _make_spmm(n_acc, h, chunks)

    y = _mm(fp, W1)
    p = spmm(y, src, dst)
    hh = _combine_relu(p)
    q = spmm(hh, src, dst)
    out = _combine_mm(q, W3)
    return out[:n]


# speedup vs baseline: 12.4997x; 12.4997x over previous
import functools

import jax
import jax.numpy as jnp
from jax import lax
from jax.experimental import pallas as pl
from jax.experimental.pallas import tpu as pltpu
from jax.experimental.pallas import tpu_sc as plsc

NC = 2
NS = 16
NW = NC * NS
L = 16
CH = 128


def _pad_up(x, m):
    return (x + m - 1) // m * m


def _make_spmm(n_acc, h, chunks):
    rows_ps = n_acc // NS
    mesh = plsc.VectorSubcoreMesh(core_axis_name="c", subcore_axis_name="s")

    @functools.partial(
        pl.kernel,
        out_type=jax.ShapeDtypeStruct((NC, n_acc, h), jnp.float32),
        mesh=mesh,
        scratch_types=[
            pltpu.VMEM((chunks, CH), jnp.int32),
            pltpu.VMEM((chunks, CH), jnp.int32),
            pltpu.VMEM((CH, h), jnp.float32),
            pltpu.VMEM((rows_ps, h), jnp.float32),
            pltpu.VMEM_SHARED((n_acc, h), jnp.float32),
            pltpu.SemaphoreType.DMA,
        ],
        compiler_params=pltpu.CompilerParams(use_tc_tiling_on_sc=False),
    )
    def spmm(table_hbm, src_hbm, dst_hbm, out_hbm,
             src_v, dst_v, rows_v, stage_v, acc_sh, sem):
        cid = lax.axis_index("c")
        sid = lax.axis_index("s")
        wid = sid * NC + cid

        def _z(i, c):
            stage_v[i, :] = jnp.zeros((L,), jnp.float32)
            return c
        lax.fori_loop(0, rows_ps, _z, 0)
        pltpu.sync_copy(stage_v, acc_sh.at[pl.ds(sid * rows_ps, rows_ps)])

        pltpu.sync_copy(src_hbm.at[wid], src_v)
        pltpu.sync_copy(dst_hbm.at[wid], dst_v)
        plsc.subcore_barrier()

        def _chunk(j, c):
            pltpu.async_copy(table_hbm.at[src_v.at[j]], rows_v, sem).wait()
            pltpu.sync_copy(rows_v, acc_sh.at[dst_v.at[j]], add=True)
            return c
        lax.fori_loop(0, chunks, _chunk, 0)
        plsc.subcore_barrier()

        pltpu.sync_copy(acc_sh.at[pl.ds(sid * rows_ps, rows_ps)], stage_v)
        pltpu.sync_copy(stage_v, out_hbm.at[cid, pl.ds(sid * rows_ps, rows_ps)])

    return spmm


def _mm(x, w):
    def body(x_ref, w_ref, o_ref):
        o_ref[...] = jnp.dot(x_ref[...], w_ref[...],
                             preferred_element_type=jnp.float32)
    return pl.pallas_call(
        body,
        out_shape=jax.ShapeDtypeStruct((x.shape[0], w.shape[1]), jnp.float32),
    )(x, w)


def _combine_relu(p):
    def body(p_ref, o_ref):
        o_ref[...] = jnp.maximum(p_ref[0] + p_ref[1], 0.0)
    return pl.pallas_call(
        body,
        out_shape=jax.ShapeDtypeStruct(p.shape[1:], jnp.float32),
    )(p)


def _combine_mm(q, w):
    def body(q_ref, w_ref, o_ref):
        o_ref[...] = jnp.dot(q_ref[0] + q_ref[1], w_ref[...],
                             preferred_element_type=jnp.float32)
    return pl.pallas_call(
        body,
        out_shape=jax.ShapeDtypeStruct((q.shape[1], w.shape[1]), jnp.float32),
    )(q, w)


def kernel(features, edge_index, W1, W3):
    n, d = features.shape
    h = W1.shape[1]
    e = edge_index.shape[1]

    n_acc = _pad_up(n + 1, NS * 8)
    chunks = _pad_up(e, NW * CH) // (NW * CH)
    e_pad = NW * chunks * CH

    fp = jnp.pad(features, ((0, n_acc - n), (0, 0)))
    pad = jnp.full((e_pad - e,), n, jnp.int32)
    src = jnp.concatenate([edge_index[0], pad]).reshape(NW, chunks, CH)
    dst = jnp.concatenate([edge_index[1], pad]).reshape(NW, chunks, CH)

    spmm = _make_spmm(n_acc, h, chunks)

    y = _mm(fp, W1)
    p = spmm(y, src, dst)
    hh = _combine_relu(p)
    q = spmm(hh, src, dst)
    out = _combine_mm(q, W3)
    return out[:n]


# speedup vs baseline: 13.3887x; 1.0711x over previous
import functools

import jax
import jax.numpy as jnp
from jax import lax
from jax.experimental import pallas as pl
from jax.experimental.pallas import tpu as pltpu
from jax.experimental.pallas import tpu_sc as plsc

NC = 2
NS = 16
NW = NC * NS
L = 16
CH = 128
K = 16


def _pad_up(x, m):
    return (x + m - 1) // m * m


def _make_spmm(n_acc, h, chunks):
    rows_ps = n_acc // NS
    mesh = plsc.VectorSubcoreMesh(core_axis_name="c", subcore_axis_name="s")

    @functools.partial(
        pl.kernel,
        out_type=jax.ShapeDtypeStruct((NC, n_acc, h), jnp.float32),
        mesh=mesh,
        scratch_types=[
            pltpu.VMEM((chunks, CH), jnp.int32),
            pltpu.VMEM((chunks, CH), jnp.int32),
            pltpu.VMEM((K, CH, h), jnp.float32),
            pltpu.VMEM((rows_ps, h), jnp.float32),
            pltpu.VMEM_SHARED((n_acc, h), jnp.float32),
            pltpu.SemaphoreType.DMA,
            pltpu.SemaphoreType.DMA,
        ],
        compiler_params=pltpu.CompilerParams(use_tc_tiling_on_sc=False),
    )
    def spmm(table_hbm, src_hbm, dst_hbm, out_hbm,
             src_v, dst_v, rows_v, stage_v, acc_sh, gsem, ssem):
        cid = lax.axis_index("c")
        sid = lax.axis_index("s")
        wid = sid * NC + cid

        def _z(i, c):
            stage_v[i, :] = jnp.zeros((L,), jnp.float32)
            return c
        lax.fori_loop(0, rows_ps, _z, 0)
        pltpu.sync_copy(stage_v, acc_sh.at[pl.ds(sid * rows_ps, rows_ps)])

        pltpu.sync_copy(src_hbm.at[wid], src_v)
        pltpu.sync_copy(dst_hbm.at[wid], dst_v)
        plsc.subcore_barrier()

        def _group(g, c):
            base = g * K
            cps = [pltpu.async_copy(table_hbm.at[src_v.at[base + b]],
                                    rows_v.at[b], gsem)
                   for b in range(K)]
            for cp in cps:
                cp.wait()
            scs = [pltpu.async_copy(rows_v.at[b], acc_sh.at[dst_v.at[base + b]],
                                    ssem, add=True)
                   for b in range(K)]
            for sc in scs:
                sc.wait()
            return c
        lax.fori_loop(0, chunks // K, _group, 0)
        plsc.subcore_barrier()

        pltpu.sync_copy(acc_sh.at[pl.ds(sid * rows_ps, rows_ps)], stage_v)
        pltpu.sync_copy(stage_v, out_hbm.at[cid, pl.ds(sid * rows_ps, rows_ps)])

    return spmm


def _mm(x, w):
    def body(x_ref, w_ref, o_ref):
        o_ref[...] = jnp.dot(x_ref[...], w_ref[...],
                             preferred_element_type=jnp.float32)
    return pl.pallas_call(
        body,
        out_shape=jax.ShapeDtypeStruct((x.shape[0], w.shape[1]), jnp.float32),
    )(x, w)


def _combine_relu(p):
    def body(p_ref, o_ref):
        o_ref[...] = jnp.maximum(p_ref[0] + p_ref[1], 0.0)
    return pl.pallas_call(
        body,
        out_shape=jax.ShapeDtypeStruct(p.shape[1:], jnp.float32),
    )(p)


def _combine_mm(q, w):
    def body(q_ref, w_ref, o_ref):
        o_ref[...] = jnp.dot(q_ref[0] + q_ref[1], w_ref[...],
                             preferred_element_type=jnp.float32)
    return pl.pallas_call(
        body,
        out_shape=jax.ShapeDtypeStruct((q.shape[1], w.shape[1]), jnp.float32),
    )(q, w)


def kernel(features, edge_index, W1, W3):
    n, d = features.shape
    h = W1.shape[1]
    e = edge_index.shape[1]

    n_acc = _pad_up(n + 1, NS * 8)
    chunks = _pad_up(_pad_up(e, NW * CH) // (NW * CH), K)
    e_pad = NW * chunks * CH

    fp = jnp.pad(features, ((0, n_acc - n), (0, 0)))
    pad = jnp.full((e_pad - e,), n, jnp.int32)
    src = jnp.concatenate([edge_index[0], pad]).reshape(NW, chunks, CH)
    dst = jnp.concatenate([edge_index[1], pad]).reshape(NW, chunks, CH)

    spmm = _make_spmm(n_acc, h, chunks)

    y = _mm(fp, W1)
    p = spmm(y, src, dst)
    hh = _combine_relu(p)
    q = spmm(hh, src, dst)
    out = _combine_mm(q, W3)
    return out[:n]


# speedup vs baseline: 21.8779x; 1.6341x over previous
import functools

import jax
import jax.numpy as jnp
from jax import lax
from jax.experimental import pallas as pl
from jax.experimental.pallas import tpu as pltpu
from jax.experimental.pallas import tpu_sc as plsc

NC = 2
NS = 16
NW = NC * NS
L = 16
CH = 128
K = 16


def _pad_up(x, m):
    return (x + m - 1) // m * m


def _make_spmm(n_acc, h, chunks):
    rows_ps = n_acc // NS
    mesh = plsc.VectorSubcoreMesh(core_axis_name="c", subcore_axis_name="s")

    @functools.partial(
        pl.kernel,
        out_type=jax.ShapeDtypeStruct((NC, n_acc, h), jnp.float32),
        mesh=mesh,
        scratch_types=[
            pltpu.VMEM((chunks, CH), jnp.int32),
            pltpu.VMEM((chunks, CH), jnp.int32),
            pltpu.VMEM((K, CH, h), jnp.float32),
            pltpu.VMEM((rows_ps, h), jnp.float32),
            pltpu.VMEM_SHARED((n_acc, h), jnp.float32),
            pltpu.VMEM_SHARED((n_acc, h), jnp.float32),
            pltpu.SemaphoreType.DMA,
            pltpu.SemaphoreType.DMA,
        ],
        compiler_params=pltpu.CompilerParams(use_tc_tiling_on_sc=False),
    )
    def spmm(table_hbm, src_hbm, dst_hbm, out_hbm,
             src_v, dst_v, rows_v, stage_v, acc_sh, table_sh, gsem, ssem):
        cid = lax.axis_index("c")
        sid = lax.axis_index("s")
        wid = sid * NC + cid

        def _z(i, c):
            stage_v[i, :] = jnp.zeros((L,), jnp.float32)
            return c
        lax.fori_loop(0, rows_ps, _z, 0)
        pltpu.sync_copy(stage_v, acc_sh.at[pl.ds(sid * rows_ps, rows_ps)])

        pltpu.sync_copy(table_hbm.at[pl.ds(sid * rows_ps, rows_ps)], stage_v)
        pltpu.sync_copy(stage_v, table_sh.at[pl.ds(sid * rows_ps, rows_ps)])

        pltpu.sync_copy(src_hbm.at[wid], src_v)
        pltpu.sync_copy(dst_hbm.at[wid], dst_v)
        plsc.subcore_barrier()

        def _group(g, c):
            base = g * K
            cps = [pltpu.async_copy(table_sh.at[src_v.at[base + b]],
                                    rows_v.at[b], gsem)
                   for b in range(K)]
            for cp in cps:
                cp.wait()
            scs = [pltpu.async_copy(rows_v.at[b], acc_sh.at[dst_v.at[base + b]],
                                    ssem, add=True)
                   for b in range(K)]
            for sc in scs:
                sc.wait()
            return c
        lax.fori_loop(0, chunks // K, _group, 0)
        plsc.subcore_barrier()

        pltpu.sync_copy(acc_sh.at[pl.ds(sid * rows_ps, rows_ps)], stage_v)
        pltpu.sync_copy(stage_v, out_hbm.at[cid, pl.ds(sid * rows_ps, rows_ps)])

    return spmm


def _mm(x, w):
    def body(x_ref, w_ref, o_ref):
        o_ref[...] = jnp.dot(x_ref[...], w_ref[...],
                             preferred_element_type=jnp.float32)
    return pl.pallas_call(
        body,
        out_shape=jax.ShapeDtypeStruct((x.shape[0], w.shape[1]), jnp.float32),
    )(x, w)


def _combine_relu(p):
    def body(p_ref, o_ref):
        o_ref[...] = jnp.maximum(p_ref[0] + p_ref[1], 0.0)
    return pl.pallas_call(
        body,
        out_shape=jax.ShapeDtypeStruct(p.shape[1:], jnp.float32),
    )(p)


def _combine_mm(q, w):
    def body(q_ref, w_ref, o_ref):
        o_ref[...] = jnp.dot(q_ref[0] + q_ref[1], w_ref[...],
                             preferred_element_type=jnp.float32)
    return pl.pallas_call(
        body,
        out_shape=jax.ShapeDtypeStruct((q.shape[1], w.shape[1]), jnp.float32),
    )(q, w)


def kernel(features, edge_index, W1, W3):
    n, d = features.shape
    h = W1.shape[1]
    e = edge_index.shape[1]

    n_acc = _pad_up(n + 1, NS * 8)
    chunks = _pad_up(_pad_up(e, NW * CH) // (NW * CH), K)
    e_pad = NW * chunks * CH

    fp = jnp.pad(features, ((0, n_acc - n), (0, 0)))
    pad = jnp.full((e_pad - e,), n, jnp.int32)
    src = jnp.concatenate([edge_index[0], pad]).reshape(NW, chunks, CH)
    dst = jnp.concatenate([edge_index[1], pad]).reshape(NW, chunks, CH)

    spmm = _make_spmm(n_acc, h, chunks)

    y = _mm(fp, W1)
    p = spmm(y, src, dst)
    hh = _combine_relu(p)
    q = spmm(hh, src, dst)
    out = _combine_mm(q, W3)
    return out[:n]


# speedup vs baseline: 27.5745x; 1.2604x over previous
import functools

import jax
import jax.numpy as jnp
from jax import lax
from jax.experimental import pallas as pl
from jax.experimental.pallas import tpu as pltpu
from jax.experimental.pallas import tpu_sc as plsc

NC = 2
NS = 16
NW = NC * NS
L = 16
CH = 128
K = 8


def _pad_up(x, m):
    return (x + m - 1) // m * m


def _make_spmm(n_tab, n_out, n_acc, h, chunks, fuse_relu):
    rows_acc = n_acc // NS
    rows_tab = n_tab // NS
    rows_out = n_out // NS
    mesh = plsc.VectorSubcoreMesh(core_axis_name="c", subcore_axis_name="s")

    @functools.partial(
        pl.kernel,
        out_type=jax.ShapeDtypeStruct((NC, n_out, h), jnp.float32),
        mesh=mesh,
        scratch_types=[
            pltpu.VMEM((chunks, CH), jnp.int32),
            pltpu.VMEM((chunks, CH), jnp.int32),
            pltpu.VMEM((K, CH, h), jnp.float32),
            pltpu.VMEM((K, CH, h), jnp.float32),
            pltpu.VMEM((rows_acc, h), jnp.float32),
            pltpu.VMEM((rows_tab, h), jnp.float32),
            pltpu.VMEM_SHARED((n_acc, h), jnp.float32),
            pltpu.VMEM_SHARED((n_acc, h), jnp.float32),
            pltpu.SemaphoreType.DMA,
            pltpu.SemaphoreType.DMA,
        ],
        compiler_params=pltpu.CompilerParams(use_tc_tiling_on_sc=False),
    )
    def spmm(table_hbm, edges_hbm, out_hbm,
             src_v, dst_v, buf_a, buf_b, stage_v, stage2_v,
             acc_sh, table_sh, gsem, ssem):
        cid = lax.axis_index("c")
        sid = lax.axis_index("s")
        wid = sid * NC + cid

        def _z(i, c):
            stage_v[i, :] = jnp.zeros((L,), jnp.float32)
            return c
        lax.fori_loop(0, rows_acc, _z, 0)
        pltpu.sync_copy(stage_v, acc_sh.at[pl.ds(sid * rows_acc, rows_acc)])

        tab = stage_v.at[pl.ds(0, rows_tab)]
        if fuse_relu:
            p2 = stage2_v.at[pl.ds(0, rows_tab)]
            pltpu.sync_copy(table_hbm.at[0, pl.ds(sid * rows_tab, rows_tab)],
                            tab)
            pltpu.sync_copy(table_hbm.at[1, pl.ds(sid * rows_tab, rows_tab)],
                            p2)

            def _relu(i, c):
                stage_v[i, :] = jnp.maximum(stage_v[i, :] + stage2_v[i, :],
                                            0.0)
                return c
            lax.fori_loop(0, rows_tab, _relu, 0)
        else:
            pltpu.sync_copy(table_hbm.at[pl.ds(sid * rows_tab, rows_tab)],
                            tab)
        pltpu.sync_copy(tab, table_sh.at[pl.ds(sid * rows_tab, rows_tab)])

        pltpu.sync_copy(edges_hbm.at[0, wid], src_v)
        pltpu.sync_copy(edges_hbm.at[1, wid], dst_v)
        plsc.subcore_barrier()

        def _pair(u, c):
            b0 = (2 * u) * K
            b1 = b0 + K
            g0 = [pltpu.async_copy(table_sh.at[src_v.at[b0 + i]],
                                   buf_a.at[i], gsem) for i in range(K)]
            g1 = [pltpu.async_copy(table_sh.at[src_v.at[b1 + i]],
                                   buf_b.at[i], gsem) for i in range(K)]
            for cp in g0:
                cp.wait()
            s0 = [pltpu.async_copy(buf_a.at[i], acc_sh.at[dst_v.at[b0 + i]],
                                   ssem, add=True) for i in range(K)]
            for cp in g1:
                cp.wait()
            s1 = [pltpu.async_copy(buf_b.at[i], acc_sh.at[dst_v.at[b1 + i]],
                                   ssem, add=True) for i in range(K)]
            for cp in s0 + s1:
                cp.wait()
            return c
        lax.fori_loop(0, chunks // (2 * K), _pair, 0)
        plsc.subcore_barrier()

        out_stage = stage_v.at[pl.ds(0, rows_out)]
        pltpu.sync_copy(acc_sh.at[pl.ds(sid * rows_out, rows_out)], out_stage)
        pltpu.sync_copy(out_stage,
                        out_hbm.at[cid, pl.ds(sid * rows_out, rows_out)])

    return spmm


def _mm1(x, w):
    def body(x_ref, w_ref, o_ref):
        o_ref[...] = jnp.dot(x_ref[...], w_ref[...],
                             preferred_element_type=jnp.float32)
    return pl.pallas_call(
        body,
        out_shape=jax.ShapeDtypeStruct((x.shape[0], w.shape[1]), jnp.float32),
    )(x, w)


def _combine_mm(q, w):
    def body(q_ref, w_ref, o_ref):
        o_ref[...] = jnp.dot(q_ref[0] + q_ref[1], w_ref[...],
                             preferred_element_type=jnp.float32)
    return pl.pallas_call(
        body,
        out_shape=jax.ShapeDtypeStruct((q.shape[1], w.shape[1]), jnp.float32),
    )(q, w)


def kernel(features, edge_index, W1, W3):
    n, d = features.shape
    h = W1.shape[1]
    e = edge_index.shape[1]

    n1 = _pad_up(n + 1, NS)
    n_acc = _pad_up(n + 1, NS * 8)
    chunks = _pad_up(_pad_up(e, NW * CH) // (NW * CH), 2 * K)
    e_pad = NW * chunks * CH

    edges = jnp.pad(edge_index, ((0, 0), (0, e_pad - e)),
                    constant_values=n).reshape(2, NW, chunks, CH)

    spmm_a = _make_spmm(n, n1, n_acc, h, chunks, fuse_relu=False)
    spmm_b = _make_spmm(n1, n, n_acc, h, chunks, fuse_relu=True)

    y = _mm1(features, W1)
    p = spmm_a(y, edges)
    q = spmm_b(p, edges)
    return _combine_mm(q, W3)


# speedup vs baseline: 30.7608x; 1.1156x over previous
import functools

import jax
import jax.numpy as jnp
from jax import lax
from jax.experimental import pallas as pl
from jax.experimental.pallas import tpu as pltpu
from jax.experimental.pallas import tpu_sc as plsc

NC = 2
NS = 16
NW = NC * NS
L = 16
CH = 128
K = 8


def _pad_up(x, m):
    return (x + m - 1) // m * m


def _make_spmm(n_tab, n_out, n_acc, h, chunks, fuse_relu):
    rows_acc = n_acc // NS
    rows_tab = n_tab // NS
    rows_out = n_out // NS
    mesh = plsc.VectorSubcoreMesh(core_axis_name="c", subcore_axis_name="s")

    @functools.partial(
        pl.kernel,
        out_type=jax.ShapeDtypeStruct((NC, n_out, h), jnp.float32),
        mesh=mesh,
        scratch_types=[
            pltpu.VMEM((chunks, CH), jnp.int32),
            pltpu.VMEM((chunks, CH), jnp.int32),
            pltpu.VMEM((K, CH, h), jnp.float32),
            pltpu.VMEM((K, CH, h), jnp.float32),
            pltpu.VMEM((rows_acc, h), jnp.float32),
            pltpu.VMEM((rows_tab, h), jnp.float32),
            pltpu.VMEM_SHARED((n_acc, h), jnp.float32),
            pltpu.VMEM_SHARED((n_acc, h), jnp.float32),
            pltpu.SemaphoreType.DMA,
            pltpu.SemaphoreType.DMA,
        ],
        compiler_params=pltpu.CompilerParams(use_tc_tiling_on_sc=False),
    )
    def spmm(table_hbm, edges_hbm, out_hbm,
             src_v, dst_v, buf_a, buf_b, stage_v, stage2_v,
             acc_sh, table_sh, gsem, ssem):
        cid = lax.axis_index("c")
        sid = lax.axis_index("s")
        wid = sid * NC + cid

        def _z(i, c):
            stage_v[i, :] = jnp.zeros((L,), jnp.float32)
            return c
        lax.fori_loop(0, rows_acc, _z, 0)
        pltpu.sync_copy(stage_v, acc_sh.at[pl.ds(sid * rows_acc, rows_acc)])

        tab = stage_v.at[pl.ds(0, rows_tab)]
        if fuse_relu:
            p2 = stage2_v.at[pl.ds(0, rows_tab)]
            pltpu.sync_copy(table_hbm.at[0, pl.ds(sid * rows_tab, rows_tab)],
                            tab)
            pltpu.sync_copy(table_hbm.at[1, pl.ds(sid * rows_tab, rows_tab)],
                            p2)

            def _relu(i, c):
                stage_v[i, :] = jnp.maximum(stage_v[i, :] + stage2_v[i, :],
                                            0.0)
                return c
            lax.fori_loop(0, rows_tab, _relu, 0)
        else:
            pltpu.sync_copy(table_hbm.at[pl.ds(sid * rows_tab, rows_tab)],
                            tab)
        pltpu.sync_copy(tab, table_sh.at[pl.ds(sid * rows_tab, rows_tab)])

        pltpu.sync_copy(edges_hbm.at[0, wid], src_v)
        pltpu.sync_copy(edges_hbm.at[1, wid], dst_v)
        plsc.subcore_barrier()

        def _pair(u, c):
            b0 = (2 * u) * K
            b1 = b0 + K
            g0 = [pltpu.async_copy(table_sh.at[src_v.at[b0 + i]],
                                   buf_a.at[i], gsem) for i in range(K)]
            g1 = [pltpu.async_copy(table_sh.at[src_v.at[b1 + i]],
                                   buf_b.at[i], gsem) for i in range(K)]
            for cp in g0:
                cp.wait()
            s0 = [pltpu.async_copy(buf_a.at[i], acc_sh.at[dst_v.at[b0 + i]],
                                   ssem, add=True) for i in range(K)]
            for cp in g1:
                cp.wait()
            s1 = [pltpu.async_copy(buf_b.at[i], acc_sh.at[dst_v.at[b1 + i]],
                                   ssem, add=True) for i in range(K)]
            for cp in s0 + s1:
                cp.wait()
            return c
        lax.fori_loop(0, chunks // (2 * K), _pair, 0)
        plsc.subcore_barrier()

        out_stage = stage_v.at[pl.ds(0, rows_out)]
        pltpu.sync_copy(acc_sh.at[pl.ds(sid * rows_out, rows_out)], out_stage)
        pltpu.sync_copy(out_stage,
                        out_hbm.at[cid, pl.ds(sid * rows_out, rows_out)])

    return spmm


def _mm1(x, wt):
    def body(x_ref, wt_ref, o_ref):
        o_ref[...] = jax.lax.dot_general(
            x_ref[...], wt_ref[...], (((1,), (1,)), ((), ())),
            preferred_element_type=jnp.float32)
    return pl.pallas_call(
        body,
        out_shape=jax.ShapeDtypeStruct((x.shape[0], wt.shape[0]), jnp.float32),
    )(x, wt)


def _combine_mm(q2, wbig, m):
    def body(q_ref, w_ref, o_ref):
        s = q_ref[0:m] + q_ref[m:2 * m]
        o_ref[...] = jnp.dot(s, w_ref[...], preferred_element_type=jnp.float32)
    return pl.pallas_call(
        body,
        out_shape=jax.ShapeDtypeStruct((m, wbig.shape[1]), jnp.float32),
    )(q2, wbig)


def kernel(features, edge_index, W1, W3):
    n, d = features.shape
    h = W1.shape[1]
    e = edge_index.shape[1]

    n1 = _pad_up(n + 1, NS)
    n_acc = _pad_up(n + 1, NS * 8)
    chunks = _pad_up(_pad_up(e, NW * CH) // (NW * CH), 2 * K)
    e_pad = NW * chunks * CH

    edges = jnp.pad(edge_index, ((0, 0), (0, e_pad - e)),
                    constant_values=n).reshape(2, NW, chunks, CH)

    spmm_a = _make_spmm(n, n1, n_acc, h, chunks, fuse_relu=False)
    spmm_b = _make_spmm(n1, n, n_acc, h, chunks, fuse_relu=True)

    y = _mm1(features, W1.T)
    p = spmm_a(y, edges)
    q = spmm_b(p, edges)

    pk = 128 // h
    m = n // pk
    q2 = q.reshape(2 * m, pk * h)
    wbig = jnp.kron(jnp.eye(pk, dtype=jnp.float32), W3)
    out2 = _combine_mm(q2, wbig, m)
    return out2.reshape(n, W3.shape[1])


# speedup vs baseline: 32.0095x; 1.0406x over previous
import functools

import jax
import jax.numpy as jnp
from jax import lax
from jax.experimental import pallas as pl
from jax.experimental.pallas import tpu as pltpu
from jax.experimental.pallas import tpu_sc as plsc

NC = 2
NS = 16
NW = NC * NS
L = 16
CH = 128
K = 8
G0 = 64
G1 = 96
GT = G0 + G1


def _pad_up(x, m):
    return (x + m - 1) // m * m


def _make_spmm(n_tab, n_out, n_acc, h, fuse_relu):
    rows_acc = n_acc // NS
    rows_tab = n_tab // NS
    rows_out = n_out // NS
    mesh = plsc.VectorSubcoreMesh(core_axis_name="c", subcore_axis_name="s")

    @functools.partial(
        pl.kernel,
        out_type=jax.ShapeDtypeStruct((NC, n_out, h), jnp.float32),
        mesh=mesh,
        scratch_types=[
            pltpu.VMEM((G1, CH), jnp.int32),
            pltpu.VMEM((G1, CH), jnp.int32),
            pltpu.VMEM((K, CH, h), jnp.float32),
            pltpu.VMEM((K, CH, h), jnp.float32),
            pltpu.VMEM((rows_acc, h), jnp.float32),
            pltpu.VMEM((rows_tab, h), jnp.float32),
            pltpu.VMEM((rows_tab, h), jnp.float32),
            pltpu.VMEM_SHARED((n_acc, h), jnp.float32),
            pltpu.VMEM_SHARED((n_acc, h), jnp.float32),
            pltpu.SemaphoreType.DMA,
            pltpu.SemaphoreType.DMA,
        ],
        compiler_params=pltpu.CompilerParams(use_tc_tiling_on_sc=False),
    )
    def spmm(table_hbm, edges_hbm, out_hbm,
             src_v, dst_v, buf_a, buf_b, zero_v, stage_v, stage2_v,
             acc_sh, table_sh, gsem, ssem):
        cid = lax.axis_index("c")
        sid = lax.axis_index("s")

        start = lax.select(cid == 0, sid * G0, NS * G0 + sid * G1)
        pairs = lax.select(cid == 0, G0 // (2 * K), G1 // (2 * K))

        cps = [pltpu.async_copy(edges_hbm.at[0, pl.ds(start, G1)], src_v,
                                gsem),
               pltpu.async_copy(edges_hbm.at[1, pl.ds(start, G1)], dst_v,
                                gsem)]
        tab_slice = pl.ds(sid * rows_tab, rows_tab)
        if fuse_relu:
            cps.append(pltpu.async_copy(table_hbm.at[0, tab_slice], stage_v,
                                        gsem))
            cps.append(pltpu.async_copy(table_hbm.at[1, tab_slice], stage2_v,
                                        gsem))
        else:
            cps.append(pltpu.async_copy(table_hbm.at[tab_slice], stage_v,
                                        gsem))

        def _z(i, c):
            zero_v[i, :] = jnp.zeros((L,), jnp.float32)
            return c
        lax.fori_loop(0, rows_acc, _z, 0, unroll=8)
        pltpu.sync_copy(zero_v, acc_sh.at[pl.ds(sid * rows_acc, rows_acc)])
        for cp in cps:
            cp.wait()

        if fuse_relu:
            def _relu(i, c):
                stage_v[i, :] = jnp.maximum(stage_v[i, :] + stage2_v[i, :],
                                            0.0)
                return c
            lax.fori_loop(0, rows_tab, _relu, 0, unroll=8)
        pltpu.sync_copy(stage_v, table_sh.at[tab_slice])
        plsc.subcore_barrier()

        def _pair(u, c):
            b0 = (2 * u) * K
            b1 = b0 + K
            g0 = [pltpu.async_copy(table_sh.at[src_v.at[b0 + i]],
                                   buf_a.at[i], gsem) for i in range(K)]
            g1 = [pltpu.async_copy(table_sh.at[src_v.at[b1 + i]],
                                   buf_b.at[i], gsem) for i in range(K)]
            for cp in g0:
                cp.wait()
            s0 = [pltpu.async_copy(buf_a.at[i], acc_sh.at[dst_v.at[b0 + i]],
                                   ssem, add=True) for i in range(K)]
            for cp in g1:
                cp.wait()
            s1 = [pltpu.async_copy(buf_b.at[i], acc_sh.at[dst_v.at[b1 + i]],
                                   ssem, add=True) for i in range(K)]
            for cp in s0 + s1:
                cp.wait()
            return c
        lax.fori_loop(0, pairs, _pair, 0)
        plsc.subcore_barrier()

        out_stage = stage_v.at[pl.ds(0, rows_out)]
        pltpu.sync_copy(acc_sh.at[pl.ds(sid * rows_out, rows_out)], out_stage)
        pltpu.sync_copy(out_stage,
                        out_hbm.at[cid, pl.ds(sid * rows_out, rows_out)])

    return spmm


def _mm1(x, wt):
    def body(x_ref, wt_ref, o_ref):
        o_ref[...] = jax.lax.dot_general(
            x_ref[...], wt_ref[...], (((1,), (1,)), ((), ())),
            preferred_element_type=jnp.float32)
    return pl.pallas_call(
        body,
        out_shape=jax.ShapeDtypeStruct((x.shape[0], wt.shape[0]), jnp.float32),
    )(x, wt)


def _combine_mm(q2, wbig, m):
    def body(q_ref, w_ref, o_ref):
        s = q_ref[0:m] + q_ref[m:2 * m]
        o_ref[...] = jnp.dot(s, w_ref[...], preferred_element_type=jnp.float32)
    return pl.pallas_call(
        body,
        out_shape=jax.ShapeDtypeStruct((m, wbig.shape[1]), jnp.float32),
    )(q2, wbig)


def kernel(features, edge_index, W1, W3):
    n, d = features.shape
    h = W1.shape[1]
    e = edge_index.shape[1]

    n1 = _pad_up(n + 1, NS)
    n_acc = _pad_up(n + 1, NS * 8)
    t_chunks = NS * GT
    e_pad = t_chunks * CH
    assert e <= e_pad

    edges = jnp.pad(edge_index, ((0, 0), (0, e_pad - e)),
                    constant_values=n).reshape(2, t_chunks, CH)

    spmm_a = _make_spmm(n, n1, n_acc, h, fuse_relu=False)
    spmm_b = _make_spmm(n1, n, n_acc, h, fuse_relu=True)

    y = _mm1(features, W1.T)
    p = spmm_a(y, edges)
    q = spmm_b(p, edges)

    pk = 128 // h
    m = n // pk
    q2 = q.reshape(2 * m, pk * h)
    wbig = jnp.kron(jnp.eye(pk, dtype=jnp.float32), W3)
    out2 = _combine_mm(q2, wbig, m)
    return out2.reshape(n, W3.shape[1])


# speedup vs baseline: 35.9603x; 1.1234x over previous
import functools

import jax
import jax.numpy as jnp
from jax import lax
from jax.experimental import pallas as pl
from jax.experimental.pallas import tpu as pltpu
from jax.experimental.pallas import tpu_sc as plsc

NC = 2
NS = 16
NW = NC * NS
L = 16
CH = 128
K = 8
G0 = 96
G1 = 64
GT = G0 + G1
GMAX = max(G0, G1)


def _pad_up(x, m):
    return (x + m - 1) // m * m


def _make_spmm(n_tab, n_out, n_acc, h, fuse_relu):
    rows_acc = n_acc // NS
    rows_tab = n_tab // NS
    rows_out = n_out // NS
    mesh = plsc.VectorSubcoreMesh(core_axis_name="c", subcore_axis_name="s")

    @functools.partial(
        pl.kernel,
        out_type=jax.ShapeDtypeStruct((NC, n_out, h), jnp.float32),
        mesh=mesh,
        scratch_types=[
            pltpu.VMEM((GMAX, CH), jnp.int32),
            pltpu.VMEM((GMAX, CH), jnp.int32),
            pltpu.VMEM((K, CH, h), jnp.float32),
            pltpu.VMEM((K, CH, h), jnp.float32),
            pltpu.VMEM((rows_acc, h), jnp.float32),
            pltpu.VMEM((rows_tab, h), jnp.float32),
            pltpu.VMEM((rows_tab, h), jnp.float32),
            pltpu.VMEM_SHARED((n_acc, h), jnp.float32),
            pltpu.VMEM_SHARED((n_acc, h), jnp.float32),
            pltpu.SemaphoreType.DMA,
            pltpu.SemaphoreType.DMA,
        ],
        compiler_params=pltpu.CompilerParams(use_tc_tiling_on_sc=False),
    )
    def spmm(table_hbm, edges_hbm, out_hbm,
             src_v, dst_v, buf_a, buf_b, zero_v, stage_v, stage2_v,
             acc_sh, table_sh, gsem, ssem):
        cid = lax.axis_index("c")
        sid = lax.axis_index("s")

        start = lax.select(cid == 0, sid * G0, NS * G0 + sid * G1)
        pairs = lax.select(cid == 0, G0 // (2 * K), G1 // (2 * K))

        cps = [pltpu.async_copy(edges_hbm.at[0, pl.ds(start, GMAX)], src_v,
                                gsem),
               pltpu.async_copy(edges_hbm.at[1, pl.ds(start, GMAX)], dst_v,
                                gsem)]
        tab_slice = pl.ds(sid * rows_tab, rows_tab)
        if fuse_relu:
            cps.append(pltpu.async_copy(table_hbm.at[0, tab_slice], stage_v,
                                        gsem))
            cps.append(pltpu.async_copy(table_hbm.at[1, tab_slice], stage2_v,
                                        gsem))
        else:
            cps.append(pltpu.async_copy(table_hbm.at[tab_slice], stage_v,
                                        gsem))

        def _z(i, c):
            zero_v[i, :] = jnp.zeros((L,), jnp.float32)
            return c
        lax.fori_loop(0, rows_acc, _z, 0, unroll=8)
        pltpu.sync_copy(zero_v, acc_sh.at[pl.ds(sid * rows_acc, rows_acc)])
        for cp in cps:
            cp.wait()

        if fuse_relu:
            def _relu(i, c):
                stage_v[i, :] = jnp.maximum(stage_v[i, :] + stage2_v[i, :],
                                            0.0)
                return c
            lax.fori_loop(0, rows_tab, _relu, 0, unroll=8)
        pltpu.sync_copy(stage_v, table_sh.at[tab_slice])
        plsc.subcore_barrier()

        def _pair(u, c):
            b0 = (2 * u) * K
            b1 = b0 + K
            g0 = [pltpu.async_copy(table_sh.at[src_v.at[b0 + i]],
                                   buf_a.at[i], gsem) for i in range(K)]
            g1 = [pltpu.async_copy(table_sh.at[src_v.at[b1 + i]],
                                   buf_b.at[i], gsem) for i in range(K)]
            for cp in g0:
                cp.wait()
            s0 = [pltpu.async_copy(buf_a.at[i], acc_sh.at[dst_v.at[b0 + i]],
                                   ssem, add=True) for i in range(K)]
            for cp in g1:
                cp.wait()
            s1 = [pltpu.async_copy(buf_b.at[i], acc_sh.at[dst_v.at[b1 + i]],
                                   ssem, add=True) for i in range(K)]
            for cp in s0 + s1:
                cp.wait()
            return c
        lax.fori_loop(0, pairs, _pair, 0)
        plsc.subcore_barrier()

        out_stage = stage_v.at[pl.ds(0, rows_out)]
        pltpu.sync_copy(acc_sh.at[pl.ds(sid * rows_out, rows_out)], out_stage)
        pltpu.sync_copy(out_stage,
                        out_hbm.at[cid, pl.ds(sid * rows_out, rows_out)])

    return spmm


def _mm1(x, wt):
    def body(x_ref, wt_ref, o_ref):
        o_ref[...] = jax.lax.dot_general(
            x_ref[...], wt_ref[...], (((1,), (1,)), ((), ())),
            preferred_element_type=jnp.float32)
    return pl.pallas_call(
        body,
        out_shape=jax.ShapeDtypeStruct((x.shape[0], wt.shape[0]), jnp.float32),
    )(x, wt)


def _combine_mm(q2, wbig, m):
    def body(q_ref, w_ref, o_ref):
        s = q_ref[0:m] + q_ref[m:2 * m]
        o_ref[...] = jnp.dot(s, w_ref[...], preferred_element_type=jnp.float32)
    return pl.pallas_call(
        body,
        out_shape=jax.ShapeDtypeStruct((m, wbig.shape[1]), jnp.float32),
    )(q2, wbig)


def kernel(features, edge_index, W1, W3):
    n, d = features.shape
    h = W1.shape[1]
    e = edge_index.shape[1]

    n1 = _pad_up(n + 1, NS)
    n_acc = _pad_up(n + 1, NS * 8)
    t_chunks = NS * GT + GMAX
    e_pad = t_chunks * CH
    assert e <= NS * GT * CH

    edges = jnp.pad(edge_index, ((0, 0), (0, e_pad - e)),
                    constant_values=n).reshape(2, t_chunks, CH)

    spmm_a = _make_spmm(n, n1, n_acc, h, fuse_relu=False)
    spmm_b = _make_spmm(n1, n, n_acc, h, fuse_relu=True)

    y = _mm1(features, W1.T)
    p = spmm_a(y, edges)
    q = spmm_b(p, edges)

    pk = 128 // h
    m = n // pk
    q2 = q.reshape(2 * m, pk * h)
    wbig = jnp.kron(jnp.eye(pk, dtype=jnp.float32), W3)
    out2 = _combine_mm(q2, wbig, m)
    return out2.reshape(n, W3.shape[1])
